# Initial kernel scaffold; baseline (speedup 1.0000x reference)
import jax
import jax.numpy as jnp
from jax.experimental import pallas as pl


def kernel(mixed_qkv, b, a, kv_cache_mem, conv_weights, dt_bias, alog, block_idx):
    raise NotImplementedError("write your pallas kernel here")


# speedup vs baseline: 1.9757x; 1.9757x over previous
import jax
import jax.numpy as jnp
from jax.experimental import pallas as pl
from jax.experimental.pallas import tpu as pltpu

HK = 4
HV = 8
DK = 128
DV = 128
KW = 4
QKV = DK * HK * 2 + DV * HV
SSM = HV * DV * DK
CONV = (KW - 1) * QKV
BLOCK = SSM + CONV
NB = 512
B = 128
LANE = 128
ROWS = BLOCK // LANE
SROWS = SSM // LANE
TROWS = QKV // LANE

_HIGH = jax.lax.Precision.HIGHEST


def _sigmoid(x):
    return 1.0 / (1.0 + jnp.exp(-x))


def _softplus(x):
    return jnp.maximum(x, 0.0) + jnp.log(1.0 + jnp.exp(-jnp.abs(x)))


def _conv_gate(mem_ref, mq_ref, cw_ref, b_ref, a_ref, dtb_ref, alog_ref):
    mq = mq_ref[0]
    c0 = mem_ref[0, SROWS:SROWS + TROWS, :]
    c1 = mem_ref[0, SROWS + TROWS:SROWS + 2 * TROWS, :]
    c2 = mem_ref[0, SROWS + 2 * TROWS:SROWS + 3 * TROWS, :]
    co = c0 * cw_ref[0] + c1 * cw_ref[1] + c2 * cw_ref[2] + mq * cw_ref[3]
    x = co * _sigmoid(co)
    g = -jnp.exp(alog_ref[:]) * _softplus(a_ref[0] + dtb_ref[:])
    beta = _sigmoid(b_ref[0])
    return x, g, beta


def _head_update(mem_ref, x, g, beta, hh):
    h = hh // (HV // HK)
    qr = x[h:h + 1, :]
    kr = x[HK + h:HK + h + 1, :]
    vr = x[2 * HK + hh:2 * HK + hh + 1, :]
    qn = qr * jax.lax.rsqrt(jnp.sum(qr * qr) + 1e-6) * (DK ** -0.5)
    kn = kr * jax.lax.rsqrt(jnp.sum(kr * kr) + 1e-6)
    S = mem_ref[0, hh * DV:(hh + 1) * DV, :]
    Sg = S * jnp.exp(g[0, hh])
    kv = jax.lax.dot_general(kn, Sg, (((1,), (1,)), ((), ())),
                             precision=_HIGH)
    delta = (vr - kv) * beta[0, hh]
    Snew = Sg + jax.lax.dot_general(delta, kn, (((0,), (0,)), ((), ())),
                                    precision=_HIGH)
    return Snew, qn


def _o_kernel(bi_ref, mem_ref, mq_ref, b_ref, a_ref, dtb_ref, alog_ref,
              cw_ref, o_ref):
    del bi_ref
    x, g, beta = _conv_gate(mem_ref, mq_ref, cw_ref, b_ref, a_ref,
                            dtb_ref, alog_ref)
    rows = []
    for hh in range(HV):
        Snew, qn = _head_update(mem_ref, x, g, beta, hh)
        rows.append(jax.lax.dot_general(qn, Snew, (((1,), (1,)), ((), ())),
                                        precision=_HIGH))
    o_ref[0] = jnp.concatenate(rows, axis=0)


def _mem_kernel(tfb_ref, mem_ref, mq_ref, b_ref, a_ref, dtb_ref, alog_ref,
                cw_ref, out_ref):
    n = pl.program_id(0)
    tok = tfb_ref[n]

    @pl.when(tok < 0)
    def _copy():
        out_ref[0] = mem_ref[0]

    @pl.when(tok >= 0)
    def _update():
        x, g, beta = _conv_gate(mem_ref, mq_ref, cw_ref, b_ref, a_ref,
                                dtb_ref, alog_ref)
        for hh in range(HV):
            Snew, _ = _head_update(mem_ref, x, g, beta, hh)
            out_ref[0, hh * DV:(hh + 1) * DV, :] = Snew
        out_ref[0, SROWS:SROWS + 2 * TROWS, :] = (
            mem_ref[0, SROWS + TROWS:SROWS + 3 * TROWS, :])
        out_ref[0, SROWS + 2 * TROWS:SROWS + 3 * TROWS, :] = mq_ref[0]


def kernel(mixed_qkv, b, a, kv_cache_mem, conv_weights, dt_bias, alog,
           block_idx):
    f32 = jnp.float32
    mem3 = kv_cache_mem.reshape(NB, ROWS, LANE)
    mq3 = mixed_qkv.reshape(B, TROWS, LANE)
    cw3 = conv_weights.T.reshape(KW, TROWS, LANE)
    b3 = b.reshape(B, 1, HV)
    a3 = a.reshape(B, 1, HV)
    dtb2 = dt_bias.reshape(1, HV)
    alog2 = alog.reshape(1, HV)
    tfb = jnp.full((NB,), -1, jnp.int32).at[block_idx].set(
        jnp.arange(B, dtype=jnp.int32))

    o = pl.pallas_call(
        _o_kernel,
        grid_spec=pltpu.PrefetchScalarGridSpec(
            num_scalar_prefetch=1,
            grid=(B,),
            in_specs=[
                pl.BlockSpec((1, ROWS, LANE), lambda i, s: (s[i], 0, 0)),
                pl.BlockSpec((1, TROWS, LANE), lambda i, s: (i, 0, 0)),
            ] + _tok_specs(lambda s, i: i),
            out_specs=pl.BlockSpec((1, HV, DV), lambda i, s: (i, 0, 0)),
        ),
        out_shape=jax.ShapeDtypeStruct((B, HV, DV), f32),
    )(block_idx, mem3, mq3, b3, a3, dtb2, alog2, cw3)

    new3 = pl.pallas_call(
        _mem_kernel,
        grid_spec=pltpu.PrefetchScalarGridSpec(
            num_scalar_prefetch=1,
            grid=(NB,),
            in_specs=[
                pl.BlockSpec((1, ROWS, LANE), lambda n, s: (n, 0, 0)),
                pl.BlockSpec((1, TROWS, LANE),
                             lambda n, s: (jnp.maximum(s[n], 0), 0, 0)),
            ] + _tok_specs(lambda s, n: jnp.maximum(s[n], 0)),
            out_specs=pl.BlockSpec((1, ROWS, LANE), lambda n, s: (n, 0, 0)),
        ),
        out_shape=jax.ShapeDtypeStruct((NB, ROWS, LANE), f32),
    )(tfb, mem3, mq3, b3, a3, dtb2, alog2, cw3)

    return o, new3.reshape(NB, BLOCK)


def _tok_specs(tok_fn):
    return [
        pl.BlockSpec((1, 1, HV), lambda i, s: (tok_fn(s, i), 0, 0)),
        pl.BlockSpec((1, 1, HV), lambda i, s: (tok_fn(s, i), 0, 0)),
        pl.BlockSpec((1, HV), lambda i, s: (0, 0)),
        pl.BlockSpec((1, HV), lambda i, s: (0, 0)),
        pl.BlockSpec((KW, TROWS, LANE), lambda i, s: (0, 0, 0)),
    ]


# speedup vs baseline: 3.5970x; 1.8206x over previous
import jax
import jax.numpy as jnp
from jax.experimental import pallas as pl
from jax.experimental.pallas import tpu as pltpu

HK = 4
HV = 8
DK = 128
DV = 128
KW = 4
QKV = DK * HK * 2 + DV * HV
SSM = HV * DV * DK
CONV = (KW - 1) * QKV
BLOCK = SSM + CONV
NB = 512
B = 128
LANE = 128
ROWS = BLOCK // LANE
SROWS = SSM // LANE
TROWS = QKV // LANE

_HIGH = jax.lax.Precision.HIGHEST


def _sigmoid(x):
    return 1.0 / (1.0 + jnp.exp(-x))


def _softplus(x):
    return jnp.maximum(x, 0.0) + jnp.log(1.0 + jnp.exp(-jnp.abs(x)))


def _conv_gate(mem_ref, mq_ref, cw_ref, b_ref, a_ref, dtb_ref, alog_ref):
    mq = mq_ref[0]
    c0 = mem_ref[0, SROWS:SROWS + TROWS, :]
    c1 = mem_ref[0, SROWS + TROWS:SROWS + 2 * TROWS, :]
    c2 = mem_ref[0, SROWS + 2 * TROWS:SROWS + 3 * TROWS, :]
    co = c0 * cw_ref[0] + c1 * cw_ref[1] + c2 * cw_ref[2] + mq * cw_ref[3]
    x = co * _sigmoid(co)
    g = -jnp.exp(alog_ref[:]) * _softplus(a_ref[0] + dtb_ref[:])
    beta = _sigmoid(b_ref[0])
    return x, g, beta


def _head_update(mem_ref, x, vcolT, g, beta, hh):
    h = hh // (HV // HK)
    qr = x[h:h + 1, :]
    kr = x[HK + h:HK + h + 1, :]
    qn = qr * jax.lax.rsqrt(jnp.sum(qr * qr) + 1e-6) * (DK ** -0.5)
    kn = kr * jax.lax.rsqrt(jnp.sum(kr * kr) + 1e-6)
    S = mem_ref[0, hh * DV:(hh + 1) * DV, :]
    Sg = S * jnp.exp(g[0, hh])
    kv = jnp.sum(Sg * kn, axis=1, keepdims=True)
    delta = (vcolT[:, hh:hh + 1] - kv) * beta[0, hh]
    Snew = Sg + delta * kn
    return Snew, qn


def _o_kernel(bi_ref, mem_ref, mq_ref, b_ref, a_ref, dtb_ref, alog_ref,
              cw_ref, o_ref):
    del bi_ref
    x, g, beta = _conv_gate(mem_ref, mq_ref, cw_ref, b_ref, a_ref,
                            dtb_ref, alog_ref)
    vcolT = x[2 * HK:, :].T
    ocols = []
    for hh in range(HV):
        Snew, qn = _head_update(mem_ref, x, vcolT, g, beta, hh)
        ocols.append(jnp.sum(Snew * qn, axis=1, keepdims=True))
    o_ref[0] = jnp.concatenate(ocols, axis=1).T


def _mem_kernel(tfb_ref, mem_ref, mq_ref, b_ref, a_ref, dtb_ref, alog_ref,
                cw_ref, out_ref):
    n = pl.program_id(0)
    tok = tfb_ref[n]

    @pl.when(tok < 0)
    def _copy():
        out_ref[0] = mem_ref[0]

    @pl.when(tok >= 0)
    def _update():
        x, g, beta = _conv_gate(mem_ref, mq_ref, cw_ref, b_ref, a_ref,
                                dtb_ref, alog_ref)
        vcolT = x[2 * HK:, :].T
        for hh in range(HV):
            Snew, _ = _head_update(mem_ref, x, vcolT, g, beta, hh)
            out_ref[0, hh * DV:(hh + 1) * DV, :] = Snew
        out_ref[0, SROWS:SROWS + 2 * TROWS, :] = (
            mem_ref[0, SROWS + TROWS:SROWS + 3 * TROWS, :])
        out_ref[0, SROWS + 2 * TROWS:SROWS + 3 * TROWS, :] = mq_ref[0]


def kernel(mixed_qkv, b, a, kv_cache_mem, conv_weights, dt_bias, alog,
           block_idx):
    f32 = jnp.float32
    mem3 = kv_cache_mem.reshape(NB, ROWS, LANE)
    mq3 = mixed_qkv.reshape(B, TROWS, LANE)
    cw3 = conv_weights.T.reshape(KW, TROWS, LANE)
    b3 = b.reshape(B, 1, HV)
    a3 = a.reshape(B, 1, HV)
    dtb2 = dt_bias.reshape(1, HV)
    alog2 = alog.reshape(1, HV)
    tfb = jnp.full((NB,), -1, jnp.int32).at[block_idx].set(
        jnp.arange(B, dtype=jnp.int32))

    o = pl.pallas_call(
        _o_kernel,
        grid_spec=pltpu.PrefetchScalarGridSpec(
            num_scalar_prefetch=1,
            grid=(B,),
            in_specs=[
                pl.BlockSpec((1, ROWS, LANE), lambda i, s: (s[i], 0, 0)),
                pl.BlockSpec((1, TROWS, LANE), lambda i, s: (i, 0, 0)),
            ] + _tok_specs(lambda s, i: i),
            out_specs=pl.BlockSpec((1, HV, DV), lambda i, s: (i, 0, 0)),
        ),
        out_shape=jax.ShapeDtypeStruct((B, HV, DV), f32),
    )(block_idx, mem3, mq3, b3, a3, dtb2, alog2, cw3)

    new3 = pl.pallas_call(
        _mem_kernel,
        grid_spec=pltpu.PrefetchScalarGridSpec(
            num_scalar_prefetch=1,
            grid=(NB,),
            in_specs=[
                pl.BlockSpec((1, ROWS, LANE), lambda n, s: (n, 0, 0)),
                pl.BlockSpec((1, TROWS, LANE),
                             lambda n, s: (jnp.maximum(s[n], 0), 0, 0)),
            ] + _tok_specs(lambda s, n: jnp.maximum(s[n], 0)),
            out_specs=pl.BlockSpec((1, ROWS, LANE), lambda n, s: (n, 0, 0)),
        ),
        out_shape=jax.ShapeDtypeStruct((NB, ROWS, LANE), f32),
    )(tfb, mem3, mq3, b3, a3, dtb2, alog2, cw3)

    return o, new3.reshape(NB, BLOCK)


def _tok_specs(tok_fn):
    return [
        pl.BlockSpec((1, 1, HV), lambda i, s: (tok_fn(s, i), 0, 0)),
        pl.BlockSpec((1, 1, HV), lambda i, s: (tok_fn(s, i), 0, 0)),
        pl.BlockSpec((1, HV), lambda i, s: (0, 0)),
        pl.BlockSpec((1, HV), lambda i, s: (0, 0)),
        pl.BlockSpec((KW, TROWS, LANE), lambda i, s: (0, 0, 0)),
    ]
